# baseline (speedup 1.0000x reference)
import jax
import jax.numpy as jnp
from jax import lax
from jax.experimental import pallas as pl
from jax.experimental.pallas import tpu as pltpu

_MAX_DET = 300
_IOU_THRES = 0.45
_CONF_THRES = 0.25
_N_BOXES = 20000
_NB = 20
_NS = 8
_NL = 128
_N_PAD = _NB * _NS * _NL


def _nms_body(xt_ref, boxes_ref, cls_ref, sco_ref,
              sc_ref, y1_ref, x1_ref, y2_ref, x2_ref, area_ref,
              ms_ref, mc_ref):
    cx = xt_ref[pl.ds(0, _NB)]
    cy = xt_ref[pl.ds(_NB, _NB)]
    w = xt_ref[pl.ds(2 * _NB, _NB)]
    h = xt_ref[pl.ds(3 * _NB, _NB)]
    obj = xt_ref[pl.ds(4 * _NB, _NB)]

    y1 = cy - h / 2.0
    x1 = cx - w / 2.0
    y2 = cy + h / 2.0
    x2 = cx + w / 2.0
    y1_ref[...] = y1
    x1_ref[...] = x1
    y2_ref[...] = y2
    x2_ref[...] = x2
    area_ref[...] = (y2 - y1) * (x2 - x1)

    m0 = xt_ref[pl.ds(5 * _NB, _NB)] * obj
    a0 = jnp.zeros_like(m0)

    def cls_step(c, carry):
        m, a = carry
        s = xt_ref[pl.ds((5 + c) * _NB, _NB)] * obj
        better = s > m
        return (jnp.where(better, s, m),
                jnp.where(better, c.astype(jnp.float32), a))

    m, a = lax.fori_loop(1, 80, cls_step, (m0, a0))
    ms_ref[...] = m
    mc_ref[...] = a
    neg = jnp.float32(-jnp.inf)
    sc_ref[...] = jnp.where(obj > _CONF_THRES, m, neg)

    shape3 = (_NB, _NS, _NL)
    biota = lax.broadcasted_iota(jnp.int32, shape3, 0)
    iota3 = (biota * (_NS * _NL)
             + lax.broadcasted_iota(jnp.int32, shape3, 1) * _NL
             + lax.broadcasted_iota(jnp.int32, shape3, 2))

    def step(t, carry):
        sc3 = sc_ref[...]
        m8 = jnp.max(sc3, axis=0)
        eq = sc3 == m8[None]
        i8 = jnp.min(jnp.where(eq, iota3, jnp.int32(2147483647)), axis=0)
        b8 = i8 // (_NS * _NL)
        oneb = biota == b8[None]

        def pick3(a3):
            return jnp.max(jnp.where(oneb, a3, neg), axis=0)

        s_v = m8
        i_v = i8
        pay = [pick3(y1_ref[...]), pick3(x1_ref[...]),
               pick3(y2_ref[...]), pick3(x2_ref[...]),
               pick3(ms_ref[...]), pick3(mc_ref[...])]

        for axis, size in ((0, _NS), (1, _NL)):
            sh = 1
            while sh < size:
                s_r = pltpu.roll(s_v, sh, axis)
                i_r = pltpu.roll(i_v, sh, axis)
                p_r = [pltpu.roll(p, sh, axis) for p in pay]
                take = (s_r > s_v) | ((s_r == s_v) & (i_r < i_v))
                s_v = jnp.where(take, s_r, s_v)
                i_v = jnp.where(take, i_r, i_v)
                pay = [jnp.where(take, r, p) for r, p in zip(p_r, pay)]
                sh *= 2

        by1, bx1, by2, bx2, bms, bmc = pay

        yy1 = jnp.maximum(by1[None], y1_ref[...])
        xx1 = jnp.maximum(bx1[None], x1_ref[...])
        yy2 = jnp.minimum(by2[None], y2_ref[...])
        xx2 = jnp.minimum(bx2[None], x2_ref[...])
        inter = jnp.maximum(yy2 - yy1, 0.0) * jnp.maximum(xx2 - xx1, 0.0)
        a1 = (by2 - by1) * (bx2 - bx1)
        iou = inter / (a1[None] + area_ref[...] - inter + 1e-9)
        supp = (iou > _IOU_THRES) | (iota3 == i_v[None])
        sc_ref[...] = jnp.where(supp, neg, sc3)

        boxes_ref[pl.ds(t, 1), pl.ds(0, 1)] = by1[0:1, 0:1]
        boxes_ref[pl.ds(t, 1), pl.ds(1, 1)] = bx1[0:1, 0:1]
        boxes_ref[pl.ds(t, 1), pl.ds(2, 1)] = by2[0:1, 0:1]
        boxes_ref[pl.ds(t, 1), pl.ds(3, 1)] = bx2[0:1, 0:1]
        cls_ref[pl.ds(t, 1), :] = bmc[0:1, 0:1]
        sco_ref[pl.ds(t, 1), :] = bms[0:1, 0:1]
        return carry

    lax.fori_loop(0, _MAX_DET, step, 0)


@jax.jit
def kernel(x):
    p = x[0]
    pad = jnp.zeros((_N_PAD - _N_BOXES, 85), jnp.float32)
    xp = jnp.concatenate([p, pad], axis=0)
    xt = xp.T.reshape(85 * _NB, _NS, _NL)
    boxes, cls, sco = pl.pallas_call(
        _nms_body,
        out_shape=[
            jax.ShapeDtypeStruct((_MAX_DET, 4), jnp.float32),
            jax.ShapeDtypeStruct((_MAX_DET, 1), jnp.float32),
            jax.ShapeDtypeStruct((_MAX_DET, 1), jnp.float32),
        ],
        scratch_shapes=[pltpu.VMEM((_NB, _NS, _NL), jnp.float32)] * 8,
    )(xt)
    return boxes[None], cls[:, 0][None], sco[:, 0][None]


# speedup vs baseline: 1.0924x; 1.0924x over previous
import jax
import jax.numpy as jnp
from jax import lax
from jax.experimental import pallas as pl
from jax.experimental.pallas import tpu as pltpu

_MAX_DET = 300
_IOU_THRES = 0.45
_CONF_THRES = 0.25
_N_BOXES = 20000
_NB = 20
_NS = 8
_NL = 128
_N_PAD = _NB * _NS * _NL
_NT = _N_PAD // _NL
_BIG = 2 ** 30


def _tree(vals, op):
    while len(vals) > 1:
        nxt = [op(vals[i], vals[i + 1]) for i in range(0, len(vals) - 1, 2)]
        if len(vals) % 2:
            nxt.append(vals[-1])
        vals = nxt
    return vals[0]


def _bcast_reduce(v, op):
    for sh in (1, 2, 4):
        v = op(v, pltpu.roll(v, sh, 0))
    w8 = _tree([v] + [pltpu.roll(v, k, 1) for k in range(1, 8)], op)
    return _tree([w8] + [pltpu.roll(w8, 8 * k, 1) for k in range(1, 16)], op)


def _slane_sum(p8):
    for sh in (1, 2, 4):
        p8 = p8 + pltpu.roll(p8, sh, 0)
    return p8


def _nms_body(x_ref, boxes_ref, cls_ref, sco_ref,
              sc_ref, y1_ref, x1_ref, y2_ref, x2_ref, area_ref,
              ms_ref, mc_ref):
    f32 = jnp.float32
    neg = f32(-jnp.inf)
    ident = (lax.broadcasted_iota(jnp.int32, (_NL, _NL), 0)
             == lax.broadcasted_iota(jnp.int32, (_NL, _NL), 1)).astype(f32)
    ones = jnp.ones((_NL, _NL), f32)
    tdn = (((0,), (0,)), ((), ()))
    sdn = (((1,), (0,)), ((), ()))

    csh = (10, _NS, _NL)
    cls_iota = (lax.broadcasted_iota(jnp.int32, csh, 0) * _NS
                + lax.broadcasted_iota(jnp.int32, csh, 1))

    def tile_one(t):
        tile = x_ref[pl.ds(t * _NL, _NL), :]
        tT = lax.dot_general(tile, ident, tdn,
                             precision=lax.Precision.HIGHEST)
        cx = tT[0:1, :]
        cy = tT[1:2, :]
        w = tT[2:3, :]
        h = tT[3:4, :]
        obj = tT[4:5, :]
        sall = (tT[5:85, :] * obj).reshape(csh)
        m8 = jnp.max(sall, axis=0)
        i8 = jnp.min(jnp.where(sall == m8[None], cls_iota, _BIG), axis=0)
        for sh in (1, 2, 4):
            m_r = pltpu.roll(m8, sh, 0)
            i_r = pltpu.roll(i8, sh, 0)
            take = (m_r > m8) | ((m_r == m8) & (i_r < i8))
            m8 = jnp.where(take, m_r, m8)
            i8 = jnp.where(take, i_r, i8)
        mrow = m8[0:1, :]
        crow = i8[0:1, :].astype(f32)

        y1 = cy - h / 2.0
        x1 = cx - w / 2.0
        y2 = cy + h / 2.0
        x2 = cx + w / 2.0
        q = t // _NS
        s = t - q * _NS

        def put(ref, val):
            ref[pl.ds(q, 1), pl.ds(s, 1), :] = val.reshape(1, 1, _NL)

        put(y1_ref, y1)
        put(x1_ref, x1)
        put(y2_ref, y2)
        put(x2_ref, x2)
        put(area_ref, (y2 - y1) * (x2 - x1))
        put(ms_ref, mrow)
        put(mc_ref, crow)
        put(sc_ref, jnp.where(obj > _CONF_THRES, mrow, neg))

    def tile_step(t4, carry):
        for j in range(4):
            tile_one(t4 * 4 + j)
        return carry

    lax.fori_loop(0, _NT // 4, tile_step, 0)

    shape3 = (_NB, _NS, _NL)
    iota3 = (lax.broadcasted_iota(jnp.int32, shape3, 0) * (_NS * _NL)
             + lax.broadcasted_iota(jnp.int32, shape3, 1) * _NL
             + lax.broadcasted_iota(jnp.int32, shape3, 2))

    def step(t, carry):
        sc3 = sc_ref[...]
        m8 = jnp.max(sc3, axis=0)
        M = _bcast_reduce(m8, jnp.maximum)
        eq3 = sc3 == M[None]

        def msum(a3, mask3):
            return _slane_sum(jnp.sum(jnp.where(mask3, a3, 0.0), axis=0))

        cnt8 = _slane_sum(jnp.sum(eq3.astype(f32), axis=0))
        stacked = jnp.concatenate(
            [cnt8,
             msum(y1_ref[...], eq3), msum(x1_ref[...], eq3),
             msum(y2_ref[...], eq3), msum(x2_ref[...], eq3),
             msum(ms_ref[...], eq3), msum(mc_ref[...], eq3)], axis=0)
        bc = lax.dot_general(stacked, ones, sdn,
                             precision=lax.Precision.HIGHEST)

        def apply(pay, oneh3):
            by1 = pay[0:8]
            bx1 = pay[8:16]
            by2 = pay[16:24]
            bx2 = pay[24:32]
            bms = pay[32:40]
            bmc = pay[40:48]
            yy1 = jnp.maximum(by1[None], y1_ref[...])
            xx1 = jnp.maximum(bx1[None], x1_ref[...])
            yy2 = jnp.minimum(by2[None], y2_ref[...])
            xx2 = jnp.minimum(bx2[None], x2_ref[...])
            inter = (jnp.maximum(yy2 - yy1, 0.0)
                     * jnp.maximum(xx2 - xx1, 0.0))
            a1 = (by2 - by1) * (bx2 - bx1)
            iou = inter / (a1[None] + area_ref[...] - inter + 1e-9)
            supp = (iou > _IOU_THRES) | oneh3
            sc_ref[...] = jnp.where(supp, neg, sc3)
            boxes_ref[pl.ds(t, 1), pl.ds(0, 1)] = by1[0:1, 0:1]
            boxes_ref[pl.ds(t, 1), pl.ds(1, 1)] = bx1[0:1, 0:1]
            boxes_ref[pl.ds(t, 1), pl.ds(2, 1)] = by2[0:1, 0:1]
            boxes_ref[pl.ds(t, 1), pl.ds(3, 1)] = bx2[0:1, 0:1]
            cls_ref[pl.ds(t, 1), :] = bmc[0:1, 0:1]
            sco_ref[pl.ds(t, 1), :] = bms[0:1, 0:1]

        apply(bc[8:56], eq3)

        @pl.when(bc[0, 0] > 1.5)
        def _fixup():
            i8 = jnp.min(jnp.where(eq3, iota3, _BIG), axis=0)
            ix = _bcast_reduce(i8, jnp.minimum)
            oneh3 = iota3 == ix[None]
            fixed = jnp.concatenate(
                [msum(y1_ref[...], oneh3), msum(x1_ref[...], oneh3),
                 msum(y2_ref[...], oneh3), msum(x2_ref[...], oneh3),
                 msum(ms_ref[...], oneh3), msum(mc_ref[...], oneh3)],
                axis=0)
            bfix = lax.dot_general(fixed, ones, sdn,
                                   precision=lax.Precision.HIGHEST)
            apply(bfix, oneh3)

        return carry

    lax.fori_loop(0, _MAX_DET, step, 0)


@jax.jit
def kernel(x):
    xp = jnp.pad(x[0], ((0, _N_PAD - _N_BOXES), (0, _NL - 85)))
    boxes, cls, sco = pl.pallas_call(
        _nms_body,
        out_shape=[
            jax.ShapeDtypeStruct((_MAX_DET, 4), jnp.float32),
            jax.ShapeDtypeStruct((_MAX_DET, 1), jnp.float32),
            jax.ShapeDtypeStruct((_MAX_DET, 1), jnp.float32),
        ],
        scratch_shapes=[pltpu.VMEM((_NB, _NS, _NL), jnp.float32)] * 8,
    )(xp)
    return boxes[None], cls[:, 0][None], sco[:, 0][None]


# speedup vs baseline: 1.2185x; 1.1154x over previous
import jax
import jax.numpy as jnp
from jax import lax
from jax.experimental import pallas as pl
from jax.experimental.pallas import tpu as pltpu

_MAX_DET = 300
_IOU_THRES = 0.45
_CONF_THRES = 0.25
_N_BOXES = 20000
_NB = 20
_NS = 8
_NL = 128
_N_PAD = _NB * _NS * _NL
_NT = _N_PAD // _NL
_BIG = 2 ** 30


def _tree(vals, op):
    while len(vals) > 1:
        nxt = [op(vals[i], vals[i + 1]) for i in range(0, len(vals) - 1, 2)]
        if len(vals) % 2:
            nxt.append(vals[-1])
        vals = nxt
    return vals[0]


def _bcast_reduce(v, op):
    for sh in (1, 2, 4):
        v = op(v, pltpu.roll(v, sh, 0))
    w8 = _tree([v] + [pltpu.roll(v, k, 1) for k in range(1, 8)], op)
    return _tree([w8] + [pltpu.roll(w8, 8 * k, 1) for k in range(1, 16)], op)


def _slane_sum(p8):
    for sh in (1, 2, 4):
        p8 = p8 + pltpu.roll(p8, sh, 0)
    return p8


def _nms_body(x_ref, boxes_ref, cls_ref, sco_ref,
              sc_ref, y1_ref, x1_ref, y2_ref, x2_ref, area_ref,
              ms_ref, mc_ref):
    f32 = jnp.float32
    neg = f32(-jnp.inf)
    ident = (lax.broadcasted_iota(jnp.int32, (_NL, _NL), 0)
             == lax.broadcasted_iota(jnp.int32, (_NL, _NL), 1)).astype(f32)
    ones = jnp.ones((_NL, _NL), f32)
    tdn = (((0,), (0,)), ((), ()))
    sdn = (((1,), (0,)), ((), ()))

    csh = (10, _NS, _NL)
    cls_iota = (lax.broadcasted_iota(jnp.int32, csh, 0) * _NS
                + lax.broadcasted_iota(jnp.int32, csh, 1))

    def tile_one(t):
        tile = x_ref[pl.ds(t * _NL, _NL), :]
        tT = lax.dot_general(tile, ident, tdn,
                             precision=lax.Precision.HIGHEST)
        cx = tT[0:1, :]
        cy = tT[1:2, :]
        w = tT[2:3, :]
        h = tT[3:4, :]
        obj = tT[4:5, :]
        sall = (tT[5:85, :] * obj).reshape(csh)
        m8 = jnp.max(sall, axis=0)
        i8 = jnp.min(jnp.where(sall == m8[None], cls_iota, _BIG), axis=0)
        for sh in (1, 2, 4):
            m_r = pltpu.roll(m8, sh, 0)
            i_r = pltpu.roll(i8, sh, 0)
            take = (m_r > m8) | ((m_r == m8) & (i_r < i8))
            m8 = jnp.where(take, m_r, m8)
            i8 = jnp.where(take, i_r, i8)
        mrow = m8[0:1, :]
        crow = i8[0:1, :].astype(f32)

        y1 = cy - h / 2.0
        x1 = cx - w / 2.0
        y2 = cy + h / 2.0
        x2 = cx + w / 2.0
        q = t // _NS
        s = t - q * _NS

        def put(ref, val):
            ref[pl.ds(q, 1), pl.ds(s, 1), :] = val.reshape(1, 1, _NL)

        put(y1_ref, y1)
        put(x1_ref, x1)
        put(y2_ref, y2)
        put(x2_ref, x2)
        put(area_ref, (y2 - y1) * (x2 - x1))
        put(ms_ref, mrow)
        put(mc_ref, crow)
        put(sc_ref, jnp.where(obj > _CONF_THRES, mrow, neg))

    def tile_step(t4, carry):
        for j in range(4):
            tile_one(t4 * 4 + j)
        return carry

    lax.fori_loop(0, _NT // 4, tile_step, 0)

    shape3 = (_NB, _NS, _NL)
    iota3 = (lax.broadcasted_iota(jnp.int32, shape3, 0) * (_NS * _NL)
             + lax.broadcasted_iota(jnp.int32, shape3, 1) * _NL
             + lax.broadcasted_iota(jnp.int32, shape3, 2))

    def step(t, sc3):
        M = jnp.max(sc3)
        eq3 = sc3 == M

        def msum(a3, mask3):
            return _slane_sum(jnp.sum(jnp.where(mask3, a3, 0.0), axis=0))

        cnt8 = _slane_sum(jnp.sum(eq3.astype(f32), axis=0))
        stacked = jnp.concatenate(
            [cnt8,
             msum(y1_ref[...], eq3), msum(x1_ref[...], eq3),
             msum(y2_ref[...], eq3), msum(x2_ref[...], eq3),
             msum(mc_ref[...], eq3)], axis=0)
        bc = lax.dot_general(stacked, ones, sdn,
                             precision=lax.Precision.HIGHEST)

        def apply(by1, bx1, by2, bx2, bms, bmc, oneh3, cur):
            yy1 = jnp.maximum(by1[None], y1_ref[...])
            xx1 = jnp.maximum(bx1[None], x1_ref[...])
            yy2 = jnp.minimum(by2[None], y2_ref[...])
            xx2 = jnp.minimum(bx2[None], x2_ref[...])
            inter = (jnp.maximum(yy2 - yy1, 0.0)
                     * jnp.maximum(xx2 - xx1, 0.0))
            a1 = (by2 - by1) * (bx2 - bx1)
            iou = inter / (a1[None] + area_ref[...] - inter + 1e-9)
            supp = (iou > _IOU_THRES) | oneh3
            newsc = jnp.where(supp, neg, cur)
            boxes_ref[pl.ds(t, 1), pl.ds(0, 1)] = by1[0:1, 0:1]
            boxes_ref[pl.ds(t, 1), pl.ds(1, 1)] = bx1[0:1, 0:1]
            boxes_ref[pl.ds(t, 1), pl.ds(2, 1)] = by2[0:1, 0:1]
            boxes_ref[pl.ds(t, 1), pl.ds(3, 1)] = bx2[0:1, 0:1]
            cls_ref[pl.ds(t, 1), :] = bmc[0:1, 0:1]
            sco_ref[pl.ds(t, 1), :] = bms[0:1, 0:1]
            return newsc

        bM = jnp.full((_NS, _NL), M, f32)
        newsc = apply(bc[8:16], bc[16:24], bc[24:32], bc[32:40],
                      bM, bc[40:48], eq3, sc3)
        sc_ref[...] = newsc

        @pl.when(bc[0, 0] > 1.5)
        def _fixup():
            i8 = jnp.min(jnp.where(eq3, iota3, _BIG), axis=0)
            ix = _bcast_reduce(i8, jnp.minimum)
            oneh3 = iota3 == ix[None]
            fixed = jnp.concatenate(
                [msum(y1_ref[...], oneh3), msum(x1_ref[...], oneh3),
                 msum(y2_ref[...], oneh3), msum(x2_ref[...], oneh3),
                 msum(ms_ref[...], oneh3), msum(mc_ref[...], oneh3)],
                axis=0)
            bfix = lax.dot_general(fixed, ones, sdn,
                                   precision=lax.Precision.HIGHEST)
            sc_ref[...] = apply(bfix[0:8], bfix[8:16], bfix[16:24],
                                bfix[24:32], bfix[32:40], bfix[40:48],
                                oneh3, sc3)

        return sc_ref[...]

    lax.fori_loop(0, _MAX_DET, step, sc_ref[...])


@jax.jit
def kernel(x):
    xp = jnp.pad(x[0], ((0, _N_PAD - _N_BOXES), (0, _NL - 85)))
    boxes, cls, sco = pl.pallas_call(
        _nms_body,
        out_shape=[
            jax.ShapeDtypeStruct((_MAX_DET, 4), jnp.float32),
            jax.ShapeDtypeStruct((_MAX_DET, 1), jnp.float32),
            jax.ShapeDtypeStruct((_MAX_DET, 1), jnp.float32),
        ],
        scratch_shapes=[pltpu.VMEM((_NB, _NS, _NL), jnp.float32)] * 8,
    )(xp)
    return boxes[None], cls[:, 0][None], sco[:, 0][None]


# speedup vs baseline: 1.2718x; 1.0438x over previous
import jax
import jax.numpy as jnp
from jax import lax
from jax.experimental import pallas as pl
from jax.experimental.pallas import tpu as pltpu

_MAX_DET = 300
_IOU_THRES = 0.45
_CONF_THRES = 0.25
_N_BOXES = 20000
_NB = 20
_NS = 8
_NL = 128
_N_PAD = _NB * _NS * _NL
_NT = _N_PAD // _NL
_BIG = 2 ** 30


def _tree(vals, op):
    while len(vals) > 1:
        nxt = [op(vals[i], vals[i + 1]) for i in range(0, len(vals) - 1, 2)]
        if len(vals) % 2:
            nxt.append(vals[-1])
        vals = nxt
    return vals[0]


def _bcast_reduce(v, op):
    for sh in (1, 2, 4):
        v = op(v, pltpu.roll(v, sh, 0))
    w8 = _tree([v] + [pltpu.roll(v, k, 1) for k in range(1, 8)], op)
    return _tree([w8] + [pltpu.roll(w8, 8 * k, 1) for k in range(1, 16)], op)


def _slane_sum(p8):
    for sh in (1, 2, 4):
        p8 = p8 + pltpu.roll(p8, sh, 0)
    return p8


def _nms_body(x_ref, boxes_ref, cls_ref, sco_ref,
              sc_ref, y1_ref, x1_ref, y2_ref, x2_ref, area_ref,
              ms_ref, mc_ref, mn_ref):
    f32 = jnp.float32
    neg = f32(-jnp.inf)
    ident = (lax.broadcasted_iota(jnp.int32, (_NL, _NL), 0)
             == lax.broadcasted_iota(jnp.int32, (_NL, _NL), 1)).astype(f32)
    ones = jnp.ones((_NL, _NL), f32)
    tdn = (((0,), (0,)), ((), ()))
    sdn = (((1,), (0,)), ((), ()))

    csh = (10, _NS, _NL)
    cls_iota = (lax.broadcasted_iota(jnp.int32, csh, 0) * _NS
                + lax.broadcasted_iota(jnp.int32, csh, 1))

    def tile_one(t):
        tile = x_ref[pl.ds(t * _NL, _NL), :]
        tT = lax.dot_general(tile, ident, tdn,
                             precision=lax.Precision.HIGHEST)
        cx = tT[0:1, :]
        cy = tT[1:2, :]
        w = tT[2:3, :]
        h = tT[3:4, :]
        obj = tT[4:5, :]
        sall = (tT[5:85, :] * obj).reshape(csh)
        m8 = jnp.max(sall, axis=0)
        i8 = jnp.min(jnp.where(sall == m8[None], cls_iota, _BIG), axis=0)
        for sh in (1, 2, 4):
            m_r = pltpu.roll(m8, sh, 0)
            i_r = pltpu.roll(i8, sh, 0)
            take = (m_r > m8) | ((m_r == m8) & (i_r < i8))
            m8 = jnp.where(take, m_r, m8)
            i8 = jnp.where(take, i_r, i8)
        mrow = m8[0:1, :]
        crow = i8[0:1, :].astype(f32)

        y1 = cy - h / 2.0
        x1 = cx - w / 2.0
        y2 = cy + h / 2.0
        x2 = cx + w / 2.0
        q = t // _NS
        s = t - q * _NS

        def put(ref, val):
            ref[pl.ds(q, 1), pl.ds(s, 1), :] = val.reshape(1, 1, _NL)

        put(y1_ref, y1)
        put(x1_ref, x1)
        put(y2_ref, y2)
        put(x2_ref, x2)
        put(area_ref, (y2 - y1) * (x2 - x1))
        put(ms_ref, mrow)
        put(mc_ref, crow)
        put(sc_ref, jnp.where(obj > _CONF_THRES, mrow, neg))

    def tile_step(t4, carry):
        for j in range(4):
            tile_one(t4 * 4 + j)
        return carry

    lax.fori_loop(0, _NT // 4, tile_step, 0)

    shape3 = (_NB, _NS, _NL)
    iota3 = (lax.broadcasted_iota(jnp.int32, shape3, 0) * (_NS * _NL)
             + lax.broadcasted_iota(jnp.int32, shape3, 1) * _NL
             + lax.broadcasted_iota(jnp.int32, shape3, 2))

    def msum(a3, mask3):
        return _slane_sum(jnp.sum(jnp.where(mask3, a3, 0.0), axis=0))[0:1]

    def step(t, M):
        sc3 = sc_ref[...]
        eq3 = sc3 == M

        cnt8 = _slane_sum(jnp.sum(eq3.astype(f32), axis=0))[0:1]
        stacked = jnp.concatenate(
            [cnt8,
             msum(y1_ref[...], eq3), msum(x1_ref[...], eq3),
             msum(y2_ref[...], eq3), msum(x2_ref[...], eq3),
             msum(mc_ref[...], eq3)], axis=0)
        bc = lax.dot_general(stacked, ones, sdn,
                             precision=lax.Precision.HIGHEST)

        def apply(by1, bx1, by2, bx2, bms, bmc, oneh3):
            yy1 = jnp.maximum(by1[None], y1_ref[...])
            xx1 = jnp.maximum(bx1[None], x1_ref[...])
            yy2 = jnp.minimum(by2[None], y2_ref[...])
            xx2 = jnp.minimum(bx2[None], x2_ref[...])
            inter = (jnp.maximum(yy2 - yy1, 0.0)
                     * jnp.maximum(xx2 - xx1, 0.0))
            a1 = (by2 - by1) * (bx2 - bx1)
            iou = inter / (a1[None] + area_ref[...] - inter + 1e-9)
            supp = (iou > _IOU_THRES) | oneh3
            newsc = jnp.where(supp, neg, sc3)
            sc_ref[...] = newsc
            mn_ref[0] = jnp.max(newsc)
            boxes_ref[pl.ds(t, 1), pl.ds(0, 1)] = by1[0:1, 0:1]
            boxes_ref[pl.ds(t, 1), pl.ds(1, 1)] = bx1[0:1, 0:1]
            boxes_ref[pl.ds(t, 1), pl.ds(2, 1)] = by2[0:1, 0:1]
            boxes_ref[pl.ds(t, 1), pl.ds(3, 1)] = bx2[0:1, 0:1]
            cls_ref[pl.ds(t, 1), :] = bmc[0:1, 0:1]
            sco_ref[pl.ds(t, 1), :] = bms[0:1, 0:1]

        bM = jnp.full((1, _NL), M, f32)
        apply(bc[1:2], bc[2:3], bc[3:4], bc[4:5], bM, bc[5:6], eq3)

        @pl.when(bc[0, 0] > 1.5)
        def _fixup():
            i8 = jnp.min(jnp.where(eq3, iota3, _BIG), axis=0)
            ix = _bcast_reduce(i8, jnp.minimum)
            oneh3 = iota3 == ix[None]
            fixed = jnp.concatenate(
                [msum(y1_ref[...], oneh3), msum(x1_ref[...], oneh3),
                 msum(y2_ref[...], oneh3), msum(x2_ref[...], oneh3),
                 msum(ms_ref[...], oneh3), msum(mc_ref[...], oneh3)],
                axis=0)
            bfix = lax.dot_general(fixed, ones, sdn,
                                   precision=lax.Precision.HIGHEST)
            apply(bfix[0:1], bfix[1:2], bfix[2:3], bfix[3:4],
                  bfix[4:5], bfix[5:6], oneh3)

        return mn_ref[0]

    lax.fori_loop(0, _MAX_DET, step, jnp.max(sc_ref[...]))


@jax.jit
def kernel(x):
    xp = jnp.pad(x[0], ((0, _N_PAD - _N_BOXES), (0, _NL - 85)))
    boxes, cls, sco = pl.pallas_call(
        _nms_body,
        out_shape=[
            jax.ShapeDtypeStruct((_MAX_DET, 4), jnp.float32),
            jax.ShapeDtypeStruct((_MAX_DET, 1), jnp.float32),
            jax.ShapeDtypeStruct((_MAX_DET, 1), jnp.float32),
        ],
        scratch_shapes=([pltpu.VMEM((_NB, _NS, _NL), jnp.float32)] * 8
                        + [pltpu.SMEM((1,), jnp.float32)]),
    )(xp)
    return boxes[None], cls[:, 0][None], sco[:, 0][None]


# speedup vs baseline: 1.2799x; 1.0063x over previous
import jax
import jax.numpy as jnp
from jax import lax
from jax.experimental import pallas as pl
from jax.experimental.pallas import tpu as pltpu

_MAX_DET = 300
_IOU_THRES = 0.45
_CONF_THRES = 0.25
_N_BOXES = 20000
_NB = 20
_NS = 8
_NL = 128
_N_PAD = _NB * _NS * _NL
_NT = _N_PAD // _NL
_BIG = 2 ** 30


def _tree(vals, op):
    while len(vals) > 1:
        nxt = [op(vals[i], vals[i + 1]) for i in range(0, len(vals) - 1, 2)]
        if len(vals) % 2:
            nxt.append(vals[-1])
        vals = nxt
    return vals[0]


def _bcast_reduce(v, op):
    for sh in (1, 2, 4):
        v = op(v, pltpu.roll(v, sh, 0))
    w8 = _tree([v] + [pltpu.roll(v, k, 1) for k in range(1, 8)], op)
    return _tree([w8] + [pltpu.roll(w8, 8 * k, 1) for k in range(1, 16)], op)


def _slane_sum(p8):
    for sh in (1, 2, 4):
        p8 = p8 + pltpu.roll(p8, sh, 0)
    return p8


def _nms_body(x_ref, boxes_ref, cls_ref, sco_ref,
              sc_ref, y1_ref, x1_ref, y2_ref, x2_ref, area_ref,
              ms_ref, mc_ref, mn_ref):
    f32 = jnp.float32
    neg = f32(-jnp.inf)
    ident = (lax.broadcasted_iota(jnp.int32, (_NL, _NL), 0)
             == lax.broadcasted_iota(jnp.int32, (_NL, _NL), 1)).astype(f32)
    ones = jnp.ones((_NL, _NL), f32)
    tdn = (((0,), (0,)), ((), ()))
    sdn = (((1,), (0,)), ((), ()))

    csh = (10, _NS, _NL)
    cls_iota = (lax.broadcasted_iota(jnp.int32, csh, 0) * _NS
                + lax.broadcasted_iota(jnp.int32, csh, 1))

    def tile_one(t):
        tile = x_ref[pl.ds(t * _NL, _NL), :]
        tT = lax.dot_general(tile, ident, tdn,
                             precision=lax.Precision.HIGHEST)
        cx = tT[0:1, :]
        cy = tT[1:2, :]
        w = tT[2:3, :]
        h = tT[3:4, :]
        obj = tT[4:5, :]
        sall = (tT[5:85, :] * obj).reshape(csh)
        m8 = jnp.max(sall, axis=0)
        i8 = jnp.min(jnp.where(sall == m8[None], cls_iota, _BIG), axis=0)
        for sh in (1, 2, 4):
            m_r = pltpu.roll(m8, sh, 0)
            i_r = pltpu.roll(i8, sh, 0)
            take = (m_r > m8) | ((m_r == m8) & (i_r < i8))
            m8 = jnp.where(take, m_r, m8)
            i8 = jnp.where(take, i_r, i8)
        mrow = m8[0:1, :]
        crow = i8[0:1, :].astype(f32)

        y1 = cy - h / 2.0
        x1 = cx - w / 2.0
        y2 = cy + h / 2.0
        x2 = cx + w / 2.0
        q = t // _NS
        s = t - q * _NS

        def put(ref, val):
            ref[pl.ds(q, 1), pl.ds(s, 1), :] = val.reshape(1, 1, _NL)

        put(y1_ref, y1)
        put(x1_ref, x1)
        put(y2_ref, y2)
        put(x2_ref, x2)
        put(area_ref, (y2 - y1) * (x2 - x1))
        put(ms_ref, mrow)
        put(mc_ref, crow)
        put(sc_ref, jnp.where(obj > _CONF_THRES, mrow, neg))

    def tile_step(t4, carry):
        for j in range(4):
            tile_one(t4 * 4 + j)
        return carry

    lax.fori_loop(0, _NT // 4, tile_step, 0)

    shape3 = (_NB, _NS, _NL)
    iota3 = (lax.broadcasted_iota(jnp.int32, shape3, 0) * (_NS * _NL)
             + lax.broadcasted_iota(jnp.int32, shape3, 1) * _NL
             + lax.broadcasted_iota(jnp.int32, shape3, 2))

    def msum(a3, mask3):
        return _slane_sum(jnp.sum(jnp.where(mask3, a3, 0.0), axis=0))[0:1]

    def step(t, M):
        sc3 = sc_ref[...]
        eq3 = sc3 == M

        cnt = jnp.sum(eq3.astype(f32))
        stacked = jnp.concatenate(
            [msum(y1_ref[...], eq3), msum(x1_ref[...], eq3),
             msum(y2_ref[...], eq3), msum(x2_ref[...], eq3),
             msum(mc_ref[...], eq3)], axis=0)
        bc = lax.dot_general(stacked, ones, sdn,
                             precision=lax.Precision.HIGHEST)

        def apply(by1, bx1, by2, bx2, bms, bmc, oneh3):
            yy1 = jnp.maximum(by1[None], y1_ref[...])
            xx1 = jnp.maximum(bx1[None], x1_ref[...])
            yy2 = jnp.minimum(by2[None], y2_ref[...])
            xx2 = jnp.minimum(bx2[None], x2_ref[...])
            inter = (jnp.maximum(yy2 - yy1, 0.0)
                     * jnp.maximum(xx2 - xx1, 0.0))
            a1 = (by2 - by1) * (bx2 - bx1)
            iou = inter / (a1[None] + area_ref[...] - inter + 1e-9)
            supp = (iou > _IOU_THRES) | oneh3
            newsc = jnp.where(supp, neg, sc3)
            sc_ref[...] = newsc
            mn_ref[0] = jnp.max(newsc)
            boxes_ref[pl.ds(t, 1), pl.ds(0, 1)] = by1[0:1, 0:1]
            boxes_ref[pl.ds(t, 1), pl.ds(1, 1)] = bx1[0:1, 0:1]
            boxes_ref[pl.ds(t, 1), pl.ds(2, 1)] = by2[0:1, 0:1]
            boxes_ref[pl.ds(t, 1), pl.ds(3, 1)] = bx2[0:1, 0:1]
            cls_ref[pl.ds(t, 1), :] = bmc[0:1, 0:1]
            sco_ref[pl.ds(t, 1), :] = bms[0:1, 0:1]

        bM = jnp.full((1, _NL), M, f32)
        apply(bc[0:1], bc[1:2], bc[2:3], bc[3:4], bM, bc[4:5], eq3)

        @pl.when(cnt > 1.5)
        def _fixup():
            i8 = jnp.min(jnp.where(eq3, iota3, _BIG), axis=0)
            ix = _bcast_reduce(i8, jnp.minimum)
            oneh3 = iota3 == ix[None]
            fixed = jnp.concatenate(
                [msum(y1_ref[...], oneh3), msum(x1_ref[...], oneh3),
                 msum(y2_ref[...], oneh3), msum(x2_ref[...], oneh3),
                 msum(ms_ref[...], oneh3), msum(mc_ref[...], oneh3)],
                axis=0)
            bfix = lax.dot_general(fixed, ones, sdn,
                                   precision=lax.Precision.HIGHEST)
            apply(bfix[0:1], bfix[1:2], bfix[2:3], bfix[3:4],
                  bfix[4:5], bfix[5:6], oneh3)

        return mn_ref[0]

    lax.fori_loop(0, _MAX_DET, step, jnp.max(sc_ref[...]))


@jax.jit
def kernel(x):
    xp = jnp.pad(x[0], ((0, _N_PAD - _N_BOXES), (0, _NL - 85)))
    boxes, cls, sco = pl.pallas_call(
        _nms_body,
        out_shape=[
            jax.ShapeDtypeStruct((_MAX_DET, 4), jnp.float32),
            jax.ShapeDtypeStruct((_MAX_DET, 1), jnp.float32),
            jax.ShapeDtypeStruct((_MAX_DET, 1), jnp.float32),
        ],
        scratch_shapes=([pltpu.VMEM((_NB, _NS, _NL), jnp.float32)] * 8
                        + [pltpu.SMEM((1,), jnp.float32)]),
    )(xp)
    return boxes[None], cls[:, 0][None], sco[:, 0][None]


# speedup vs baseline: 1.2805x; 1.0005x over previous
import jax
import jax.numpy as jnp
from jax import lax
from jax.experimental import pallas as pl
from jax.experimental.pallas import tpu as pltpu

_MAX_DET = 300
_IOU_THRES = 0.45
_CONF_THRES = 0.25
_N_BOXES = 20000
_NB = 20
_NS = 8
_NL = 128
_N_PAD = _NB * _NS * _NL
_NT = _N_PAD // _NL
_BIG = 2 ** 30


def _tree(vals, op):
    while len(vals) > 1:
        nxt = [op(vals[i], vals[i + 1]) for i in range(0, len(vals) - 1, 2)]
        if len(vals) % 2:
            nxt.append(vals[-1])
        vals = nxt
    return vals[0]


def _bcast_reduce(v, op):
    for sh in (1, 2, 4):
        v = op(v, pltpu.roll(v, sh, 0))
    w8 = _tree([v] + [pltpu.roll(v, k, 1) for k in range(1, 8)], op)
    return _tree([w8] + [pltpu.roll(w8, 8 * k, 1) for k in range(1, 16)], op)


def _slane_sum(p8):
    for sh in (1, 2, 4):
        p8 = p8 + pltpu.roll(p8, sh, 0)
    return p8


def _nms_body(x_ref, boxes_ref, cls_ref, sco_ref,
              sc_ref, y1_ref, x1_ref, y2_ref, x2_ref, area_ref,
              ms_ref, mc_ref, mn_ref):
    f32 = jnp.float32
    neg = f32(-jnp.inf)
    ident = (lax.broadcasted_iota(jnp.int32, (_NL, _NL), 0)
             == lax.broadcasted_iota(jnp.int32, (_NL, _NL), 1)).astype(f32)
    ones = jnp.ones((_NL, _NL), f32)
    tdn = (((0,), (0,)), ((), ()))
    sdn = (((1,), (0,)), ((), ()))

    csh = (10, _NS, _NL)
    cls_iota = (lax.broadcasted_iota(jnp.int32, csh, 0) * _NS
                + lax.broadcasted_iota(jnp.int32, csh, 1))

    def tile_one(t):
        tile = x_ref[pl.ds(t * _NL, _NL), :]
        tT = lax.dot_general(tile, ident, tdn,
                             precision=lax.Precision.HIGHEST)
        cx = tT[0:1, :]
        cy = tT[1:2, :]
        w = tT[2:3, :]
        h = tT[3:4, :]
        obj = tT[4:5, :]
        sall = (tT[5:85, :] * obj).reshape(csh)
        m8 = jnp.max(sall, axis=0)
        i8 = jnp.min(jnp.where(sall == m8[None], cls_iota, _BIG), axis=0)
        for sh in (1, 2, 4):
            m_r = pltpu.roll(m8, sh, 0)
            i_r = pltpu.roll(i8, sh, 0)
            take = (m_r > m8) | ((m_r == m8) & (i_r < i8))
            m8 = jnp.where(take, m_r, m8)
            i8 = jnp.where(take, i_r, i8)
        mrow = m8[0:1, :]
        crow = i8[0:1, :].astype(f32)

        y1 = cy - h / 2.0
        x1 = cx - w / 2.0
        y2 = cy + h / 2.0
        x2 = cx + w / 2.0
        q = t // _NS
        s = t - q * _NS

        def put(ref, val):
            ref[pl.ds(q, 1), pl.ds(s, 1), :] = val.reshape(1, 1, _NL)

        put(y1_ref, y1)
        put(x1_ref, x1)
        put(y2_ref, y2)
        put(x2_ref, x2)
        put(area_ref, (y2 - y1) * (x2 - x1))
        put(ms_ref, mrow)
        put(mc_ref, crow)
        put(sc_ref, jnp.where(obj > _CONF_THRES, mrow, neg))

    def tile_step(t4, carry):
        for j in range(4):
            tile_one(t4 * 4 + j)
        return carry

    lax.fori_loop(0, _NT // 4, tile_step, 0)

    shape3 = (_NB, _NS, _NL)
    iota3 = (lax.broadcasted_iota(jnp.int32, shape3, 0) * (_NS * _NL)
             + lax.broadcasted_iota(jnp.int32, shape3, 1) * _NL
             + lax.broadcasted_iota(jnp.int32, shape3, 2))

    def msum(a3, mask3):
        return _slane_sum(jnp.sum(jnp.where(mask3, a3, 0.0), axis=0))[0:1]

    def pick(t, M, y1a, x1a, y2a, x2a, ara, mca):
        sc3 = sc_ref[...]
        eq3 = sc3 == M

        cnt = jnp.sum(eq3.astype(f32))
        stacked = jnp.concatenate(
            [msum(y1a, eq3), msum(x1a, eq3),
             msum(y2a, eq3), msum(x2a, eq3),
             msum(mca, eq3)], axis=0)
        bc = lax.dot_general(stacked, ones, sdn,
                             precision=lax.Precision.HIGHEST)

        def apply(by1, bx1, by2, bx2, bms, bmc, oneh3):
            yy1 = jnp.maximum(by1[None], y1a)
            xx1 = jnp.maximum(bx1[None], x1a)
            yy2 = jnp.minimum(by2[None], y2a)
            xx2 = jnp.minimum(bx2[None], x2a)
            inter = (jnp.maximum(yy2 - yy1, 0.0)
                     * jnp.maximum(xx2 - xx1, 0.0))
            a1 = (by2 - by1) * (bx2 - bx1)
            iou = inter / (a1[None] + ara - inter + 1e-9)
            supp = (iou > _IOU_THRES) | oneh3
            newsc = jnp.where(supp, neg, sc3)
            sc_ref[...] = newsc
            mn_ref[0] = jnp.max(newsc)
            boxes_ref[pl.ds(t, 1), pl.ds(0, 1)] = by1[0:1, 0:1]
            boxes_ref[pl.ds(t, 1), pl.ds(1, 1)] = bx1[0:1, 0:1]
            boxes_ref[pl.ds(t, 1), pl.ds(2, 1)] = by2[0:1, 0:1]
            boxes_ref[pl.ds(t, 1), pl.ds(3, 1)] = bx2[0:1, 0:1]
            cls_ref[pl.ds(t, 1), :] = bmc[0:1, 0:1]
            sco_ref[pl.ds(t, 1), :] = bms[0:1, 0:1]

        bM = jnp.full((1, _NL), M, f32)
        apply(bc[0:1], bc[1:2], bc[2:3], bc[3:4], bM, bc[4:5], eq3)

        @pl.when(cnt > 1.5)
        def _fixup():
            i8 = jnp.min(jnp.where(eq3, iota3, _BIG), axis=0)
            ix = _bcast_reduce(i8, jnp.minimum)
            oneh3 = iota3 == ix[None]
            fixed = jnp.concatenate(
                [msum(y1a, oneh3), msum(x1a, oneh3),
                 msum(y2a, oneh3), msum(x2a, oneh3),
                 msum(ms_ref[...], oneh3), msum(mca, oneh3)],
                axis=0)
            bfix = lax.dot_general(fixed, ones, sdn,
                                   precision=lax.Precision.HIGHEST)
            apply(bfix[0:1], bfix[1:2], bfix[2:3], bfix[3:4],
                  bfix[4:5], bfix[5:6], oneh3)

        return mn_ref[0]

    def pair(u, M):
        y1a = y1_ref[...]
        x1a = x1_ref[...]
        y2a = y2_ref[...]
        x2a = x2_ref[...]
        ara = area_ref[...]
        mca = mc_ref[...]
        M1 = pick(2 * u, M, y1a, x1a, y2a, x2a, ara, mca)
        return pick(2 * u + 1, M1, y1a, x1a, y2a, x2a, ara, mca)

    lax.fori_loop(0, _MAX_DET // 2, pair, jnp.max(sc_ref[...]))


@jax.jit
def kernel(x):
    xp = jnp.pad(x[0], ((0, _N_PAD - _N_BOXES), (0, _NL - 85)))
    boxes, cls, sco = pl.pallas_call(
        _nms_body,
        out_shape=[
            jax.ShapeDtypeStruct((_MAX_DET, 4), jnp.float32),
            jax.ShapeDtypeStruct((_MAX_DET, 1), jnp.float32),
            jax.ShapeDtypeStruct((_MAX_DET, 1), jnp.float32),
        ],
        scratch_shapes=([pltpu.VMEM((_NB, _NS, _NL), jnp.float32)] * 8
                        + [pltpu.SMEM((1,), jnp.float32)]),
    )(xp)
    return boxes[None], cls[:, 0][None], sco[:, 0][None]


# speedup vs baseline: 1.4467x; 1.1298x over previous
import jax
import jax.numpy as jnp
from jax import lax
from jax.experimental import pallas as pl
from jax.experimental.pallas import tpu as pltpu

_MAX_DET = 300
_IOU_THRES = 0.45
_CONF_THRES = 0.25
_N_BOXES = 20000
_NB = 20
_NS = 8
_NL = 128
_N_PAD = _NB * _NS * _NL
_NT = _N_PAD // _NL
_BIG = 2 ** 30


def _tree(vals, op):
    while len(vals) > 1:
        nxt = [op(vals[i], vals[i + 1]) for i in range(0, len(vals) - 1, 2)]
        if len(vals) % 2:
            nxt.append(vals[-1])
        vals = nxt
    return vals[0]


def _bcast_reduce(v, op):
    for sh in (1, 2, 4):
        v = op(v, pltpu.roll(v, sh, 0))
    w8 = _tree([v] + [pltpu.roll(v, k, 1) for k in range(1, 8)], op)
    return _tree([w8] + [pltpu.roll(w8, 8 * k, 1) for k in range(1, 16)], op)


def _slane_sum(p8):
    for sh in (1, 2, 4):
        p8 = p8 + pltpu.roll(p8, sh, 0)
    return p8


def _nms_body(x_ref, boxes_ref, cls_ref, sco_ref,
              sc_ref, y1_ref, x1_ref, y2_ref, x2_ref, area_ref,
              ms_ref, mc_ref, mn_ref):
    f32 = jnp.float32
    neg = f32(-jnp.inf)
    ident = (lax.broadcasted_iota(jnp.int32, (_NL, _NL), 0)
             == lax.broadcasted_iota(jnp.int32, (_NL, _NL), 1)).astype(f32)
    ones = jnp.ones((_NL, _NL), f32)
    tdn = (((0,), (0,)), ((), ()))
    sdn = (((1,), (0,)), ((), ()))

    csh = (10, _NS, _NL)
    cls_iota = (lax.broadcasted_iota(jnp.int32, csh, 0) * _NS
                + lax.broadcasted_iota(jnp.int32, csh, 1))

    def tile_one(t):
        tile = x_ref[pl.ds(t * _NL, _NL), :]
        tT = lax.dot_general(tile, ident, tdn,
                             precision=lax.Precision.HIGHEST)
        cx = tT[0:1, :]
        cy = tT[1:2, :]
        w = tT[2:3, :]
        h = tT[3:4, :]
        obj = tT[4:5, :]
        sall = (tT[5:85, :] * obj).reshape(csh)
        m8 = jnp.max(sall, axis=0)
        i8 = jnp.min(jnp.where(sall == m8[None], cls_iota, _BIG), axis=0)
        for sh in (1, 2, 4):
            m_r = pltpu.roll(m8, sh, 0)
            i_r = pltpu.roll(i8, sh, 0)
            take = (m_r > m8) | ((m_r == m8) & (i_r < i8))
            m8 = jnp.where(take, m_r, m8)
            i8 = jnp.where(take, i_r, i8)
        mrow = m8[0:1, :]
        crow = i8[0:1, :].astype(f32)

        y1 = cy - h / 2.0
        x1 = cx - w / 2.0
        y2 = cy + h / 2.0
        x2 = cx + w / 2.0
        q = t // _NS
        s = t - q * _NS

        def put(ref, val):
            ref[pl.ds(q, 1), pl.ds(s, 1), :] = val.reshape(1, 1, _NL)

        put(y1_ref, y1)
        put(x1_ref, x1)
        put(y2_ref, y2)
        put(x2_ref, x2)
        put(area_ref, (y2 - y1) * (x2 - x1))
        put(ms_ref, mrow)
        put(mc_ref, crow)
        put(sc_ref, jnp.where(obj > _CONF_THRES, mrow, neg))

    def tile_step(t4, carry):
        for j in range(4):
            tile_one(t4 * 4 + j)
        return carry

    lax.fori_loop(0, _NT // 4, tile_step, 0)

    shape3 = (_NB, _NS, _NL)
    iota3 = (lax.broadcasted_iota(jnp.int32, shape3, 0) * (_NS * _NL)
             + lax.broadcasted_iota(jnp.int32, shape3, 1) * _NL
             + lax.broadcasted_iota(jnp.int32, shape3, 2))

    def msum(a3, mask3):
        return _slane_sum(jnp.sum(jnp.where(mask3, a3, 0.0), axis=0))[0:1]

    def pick(t, M, sc3, y1a, x1a, y2a, x2a, ara, mca):
        eq3 = sc3 == M

        cnt = jnp.sum(eq3.astype(f32))
        stacked = jnp.concatenate(
            [msum(y1a, eq3), msum(x1a, eq3),
             msum(y2a, eq3), msum(x2a, eq3),
             msum(mca, eq3)], axis=0)
        bc = lax.dot_general(stacked, ones, sdn,
                             precision=lax.Precision.HIGHEST)

        def apply(by1, bx1, by2, bx2, bms, bmc, oneh3):
            yy1 = jnp.maximum(by1[None], y1a)
            xx1 = jnp.maximum(bx1[None], x1a)
            yy2 = jnp.minimum(by2[None], y2a)
            xx2 = jnp.minimum(bx2[None], x2a)
            inter = (jnp.maximum(yy2 - yy1, 0.0)
                     * jnp.maximum(xx2 - xx1, 0.0))
            a1 = (by2 - by1) * (bx2 - bx1)
            iou = inter / (a1[None] + ara - inter + 1e-9)
            supp = (iou > _IOU_THRES) | oneh3
            newsc = jnp.where(supp, neg, sc3)
            sc_ref[...] = newsc
            mn_ref[0] = jnp.max(newsc)
            boxes_ref[pl.ds(t, 1), pl.ds(0, 1)] = by1[0:1, 0:1]
            boxes_ref[pl.ds(t, 1), pl.ds(1, 1)] = bx1[0:1, 0:1]
            boxes_ref[pl.ds(t, 1), pl.ds(2, 1)] = by2[0:1, 0:1]
            boxes_ref[pl.ds(t, 1), pl.ds(3, 1)] = bx2[0:1, 0:1]
            cls_ref[pl.ds(t, 1), :] = bmc[0:1, 0:1]
            sco_ref[pl.ds(t, 1), :] = bms[0:1, 0:1]

        bM = jnp.full((1, _NL), M, f32)
        apply(bc[0:1], bc[1:2], bc[2:3], bc[3:4], bM, bc[4:5], eq3)

        @pl.when(cnt > 1.5)
        def _fixup():
            i8 = jnp.min(jnp.where(eq3, iota3, _BIG), axis=0)
            ix = _bcast_reduce(i8, jnp.minimum)
            oneh3 = iota3 == ix[None]
            fixed = jnp.concatenate(
                [msum(y1a, oneh3), msum(x1a, oneh3),
                 msum(y2a, oneh3), msum(x2a, oneh3),
                 msum(ms_ref[...], oneh3), msum(mca, oneh3)],
                axis=0)
            bfix = lax.dot_general(fixed, ones, sdn,
                                   precision=lax.Precision.HIGHEST)
            apply(bfix[0:1], bfix[1:2], bfix[2:3], bfix[3:4],
                  bfix[4:5], bfix[5:6], oneh3)

        return mn_ref[0]

    def pair(u, M):
        y1a = y1_ref[...]
        x1a = x1_ref[...]
        y2a = y2_ref[...]
        x2a = x2_ref[...]
        ara = area_ref[...]
        mca = mc_ref[...]
        sc3 = sc_ref[...]
        t = 2 * u

        eq1 = sc3 == M
        cnt1 = jnp.sum(eq1.astype(f32))
        M2 = jnp.max(jnp.where(eq1, neg, sc3))
        eq2 = sc3 == M2
        cnt2 = jnp.sum(eq2.astype(f32))
        stacked = jnp.concatenate(
            [msum(y1a, eq1), msum(x1a, eq1), msum(y2a, eq1),
             msum(x2a, eq1), msum(mca, eq1),
             msum(y1a, eq2), msum(x1a, eq2), msum(y2a, eq2),
             msum(x2a, eq2), msum(mca, eq2)], axis=0)
        bc = lax.dot_general(stacked, ones, sdn,
                             precision=lax.Precision.HIGHEST)
        c1y1, c1x1, c1y2, c1x2 = bc[0:1], bc[1:2], bc[2:3], bc[3:4]
        c2y1, c2x1, c2y2, c2x2 = bc[5:6], bc[6:7], bc[7:8], bc[8:9]
        ry1 = jnp.maximum(c1y1, c2y1)
        rx1 = jnp.maximum(c1x1, c2x1)
        ry2 = jnp.minimum(c1y2, c2y2)
        rx2 = jnp.minimum(c1x2, c2x2)
        rint = (jnp.maximum(ry2 - ry1, 0.0)
                * jnp.maximum(rx2 - rx1, 0.0))
        a1r = (c1y2 - c1y1) * (c1x2 - c1x1)
        a2r = (c2y2 - c2y1) * (c2x2 - c2x1)
        iou12 = rint / (a1r + a2r - rint + 1e-9)
        bad = ((cnt1 > 1.5) | (cnt2 > 1.5)
               | (iou12[0, 0] > _IOU_THRES))

        yy1 = jnp.maximum(c1y1[None], y1a)
        xx1 = jnp.maximum(c1x1[None], x1a)
        yy2 = jnp.minimum(c1y2[None], y2a)
        xx2 = jnp.minimum(c1x2[None], x2a)
        int1 = jnp.maximum(yy2 - yy1, 0.0) * jnp.maximum(xx2 - xx1, 0.0)
        iou1 = int1 / (a1r[None] + ara - int1 + 1e-9)
        zz1 = jnp.maximum(c2y1[None], y1a)
        zx1 = jnp.maximum(c2x1[None], x1a)
        zz2 = jnp.minimum(c2y2[None], y2a)
        zx2 = jnp.minimum(c2x2[None], x2a)
        int2 = jnp.maximum(zz2 - zz1, 0.0) * jnp.maximum(zx2 - zx1, 0.0)
        iou2 = int2 / (a2r[None] + ara - int2 + 1e-9)
        supp = ((iou1 > _IOU_THRES) | (iou2 > _IOU_THRES) | eq1 | eq2)
        newsc = jnp.where(supp, neg, sc3)
        sc_ref[...] = newsc
        mn_ref[0] = jnp.max(newsc)
        boxes_ref[pl.ds(t, 1), pl.ds(0, 1)] = c1y1[0:1, 0:1]
        boxes_ref[pl.ds(t, 1), pl.ds(1, 1)] = c1x1[0:1, 0:1]
        boxes_ref[pl.ds(t, 1), pl.ds(2, 1)] = c1y2[0:1, 0:1]
        boxes_ref[pl.ds(t, 1), pl.ds(3, 1)] = c1x2[0:1, 0:1]
        cls_ref[pl.ds(t, 1), :] = bc[4:5][0:1, 0:1]
        sco_ref[pl.ds(t, 1), :] = jnp.full((1, 1), M, f32)
        boxes_ref[pl.ds(t + 1, 1), pl.ds(0, 1)] = c2y1[0:1, 0:1]
        boxes_ref[pl.ds(t + 1, 1), pl.ds(1, 1)] = c2x1[0:1, 0:1]
        boxes_ref[pl.ds(t + 1, 1), pl.ds(2, 1)] = c2y2[0:1, 0:1]
        boxes_ref[pl.ds(t + 1, 1), pl.ds(3, 1)] = c2x2[0:1, 0:1]
        cls_ref[pl.ds(t + 1, 1), :] = bc[9:10][0:1, 0:1]
        sco_ref[pl.ds(t + 1, 1), :] = jnp.full((1, 1), M2, f32)

        @pl.when(bad)
        def _slow():
            M1 = pick(t, M, sc3, y1a, x1a, y2a, x2a, ara, mca)
            pick(t + 1, M1, sc_ref[...], y1a, x1a, y2a, x2a, ara, mca)

        return mn_ref[0]

    lax.fori_loop(0, _MAX_DET // 2, pair, jnp.max(sc_ref[...]))


@jax.jit
def kernel(x):
    xp = jnp.pad(x[0], ((0, _N_PAD - _N_BOXES), (0, _NL - 85)))
    boxes, cls, sco = pl.pallas_call(
        _nms_body,
        out_shape=[
            jax.ShapeDtypeStruct((_MAX_DET, 4), jnp.float32),
            jax.ShapeDtypeStruct((_MAX_DET, 1), jnp.float32),
            jax.ShapeDtypeStruct((_MAX_DET, 1), jnp.float32),
        ],
        scratch_shapes=([pltpu.VMEM((_NB, _NS, _NL), jnp.float32)] * 8
                        + [pltpu.SMEM((1,), jnp.float32)]),
    )(xp)
    return boxes[None], cls[:, 0][None], sco[:, 0][None]
